# baseline (speedup 1.0000x reference)
import functools

import jax
import jax.numpy as jnp
from jax.experimental import pallas as pl

N = 16384
M = 1024
C = 128
H = 256
TN = 512


def _half(v, b, f, w1, b1, w2, b2):
    vv = jnp.sum(v * v, axis=1, keepdims=True)
    ones_col = jnp.ones_like(vv)
    a = jnp.concatenate([v * -2.0, vv, ones_col], axis=1)
    d2 = jnp.dot(a, b, preferred_element_type=jnp.float32,
                 precision=jax.lax.Precision.HIGHEST)
    d2 = jnp.maximum(d2, 0.0)

    inf = jnp.float32(jnp.inf)
    m0 = jnp.min(d2, axis=1, keepdims=True)
    d2a = jnp.where(d2 > m0, d2, inf)
    m1 = jnp.min(d2a, axis=1, keepdims=True)
    d2b = jnp.where(d2a > m1, d2a, inf)
    m2 = jnp.min(d2b, axis=1, keepdims=True)

    w0 = 1.0 / m0
    has_inf = jnp.isinf(w0)
    thresh = jnp.where(has_inf, m0, m2)
    wsum = jnp.where(has_inf, 1.0, w0 + 1.0 / m1 + 1.0 / m2)

    recip = 1.0 / d2
    val = jnp.where(has_inf, 1.0, recip)
    wmat = jnp.where(d2 <= thresh, val, 0.0)

    interp = jnp.dot(wmat, f, preferred_element_type=jnp.float32)
    interp = interp * (1.0 / wsum)

    h = jnp.dot(interp, w1, preferred_element_type=jnp.float32)
    h = jnp.maximum(h + b1, 0.0)
    out = jnp.dot(h, w2, preferred_element_type=jnp.float32)
    return out + b2


def _fused_body(v_ref, ct_ref, f_ref, w1_ref, b1_ref, w2_ref, b2_ref, o_ref):
    ct = ct_ref[...]
    c2 = jnp.sum(ct * ct, axis=0, keepdims=True)
    b = jnp.concatenate([ct, jnp.ones_like(c2), c2], axis=0)
    f = f_ref[...]
    w1 = w1_ref[...]
    b1 = b1_ref[...]
    w2 = w2_ref[...]
    b2 = b2_ref[...]
    th = TN // 2
    o_ref[pl.ds(0, th), :] = _half(
        v_ref[pl.ds(0, th), :], b, f, w1, b1, w2, b2)
    o_ref[pl.ds(th, th), :] = _half(
        v_ref[pl.ds(th, th), :], b, f, w1, b1, w2, b2)


@functools.partial(jax.jit, static_argnames=())
def kernel(vertices, centroids, centroid_features, W1, b1, W2, b2):
    ct = centroids.T
    w1a = W1[:C, :]
    b1r = b1.reshape(1, H)
    b2r = b2.reshape(1, H)
    grid = (N // TN,)
    out = pl.pallas_call(
        _fused_body,
        grid=grid,
        in_specs=[
            pl.BlockSpec((TN, 3), lambda i: (i, 0)),
            pl.BlockSpec((3, M), lambda i: (0, 0)),
            pl.BlockSpec((M, C), lambda i: (0, 0)),
            pl.BlockSpec((C, H), lambda i: (0, 0)),
            pl.BlockSpec((1, H), lambda i: (0, 0)),
            pl.BlockSpec((H, H), lambda i: (0, 0)),
            pl.BlockSpec((1, H), lambda i: (0, 0)),
        ],
        out_specs=pl.BlockSpec((TN, H), lambda i: (i, 0)),
        out_shape=jax.ShapeDtypeStruct((N, H), jnp.float32),
    )(vertices, ct, centroid_features, w1a, b1r, W2, b2r)
    return out


# speedup vs baseline: 1.1359x; 1.1359x over previous
import functools

import jax
import jax.numpy as jnp
from jax.experimental import pallas as pl

N = 16384
M = 1024
C = 128
H = 256
TN = 1024
WCLAMP = 1e30


def _merge22(a1, a2, b1, b2):
    c1 = jnp.minimum(a1, b1)
    x = jnp.maximum(a1, b1)
    y = jnp.minimum(a2, b2)
    c2 = jnp.minimum(x, y)
    c3 = jnp.minimum(jnp.maximum(x, y), jnp.maximum(a2, b2))
    return c1, c2, c3


def _merge33(a, b):
    a1, a2, a3 = a
    b1, b2, b3 = b
    c1 = jnp.minimum(a1, b1)
    y1 = jnp.maximum(a1, b1)
    x2 = jnp.minimum(a2, b2)
    y2 = jnp.maximum(a2, b2)
    x3 = jnp.minimum(a3, b3)
    c2 = jnp.minimum(y1, x2)
    z = jnp.maximum(y1, x2)
    c3 = jnp.minimum(jnp.minimum(z, x3), y2)
    return c1, c2, c3


def _third_smallest(d2):
    inf = jnp.float32(jnp.inf)
    half = M // 2
    s1 = jnp.minimum(d2[:half, :], d2[half:, :])
    s2 = jnp.maximum(d2[:half, :], d2[half:, :])
    h = half // 2
    a = _merge22(s1[:h, :], s2[:h, :], s1[h:, :], s2[h:, :])
    rows = h
    while rows > 8:
        h = rows // 2
        a = _merge33(tuple(x[:h, :] for x in a), tuple(x[h:, :] for x in a))
        rows = h
    a1, a2, a3 = a
    m0 = jnp.min(a1, axis=0, keepdims=True)
    c1 = jnp.where(a1 > m0, a1, inf)
    m1 = jnp.minimum(jnp.min(c1, axis=0, keepdims=True),
                     jnp.min(a2, axis=0, keepdims=True))
    c1b = jnp.where(c1 > m1, c1, inf)
    c2b = jnp.where(a2 > m1, a2, inf)
    m2 = jnp.minimum(
        jnp.minimum(jnp.min(c1b, axis=0, keepdims=True),
                    jnp.min(c2b, axis=0, keepdims=True)),
        jnp.min(a3, axis=0, keepdims=True))
    return m2


def _fused_body(vt_ref, cx_ref, cy_ref, cz_ref, f_ref, w1_ref, b1_ref,
                w2_ref, b2_ref, o_ref):
    vt = vt_ref[...]

    dx = cx_ref[...] - vt[0:1, :]
    d2 = dx * dx
    dy = cy_ref[...] - vt[1:2, :]
    d2 += dy * dy
    dz = cz_ref[...] - vt[2:3, :]
    d2 += dz * dz

    m2 = _third_smallest(d2)

    d2sel = jnp.where(d2 <= m2, d2, jnp.float32(jnp.inf))
    wmat = jnp.minimum(1.0 / d2sel, WCLAMP)

    acc = jax.lax.dot_general(
        wmat, f_ref[...], (((0,), (0,)), ((), ())),
        preferred_element_type=jnp.float32)
    interp = acc[:, :C] / acc[:, C:C + 1]

    h = jnp.dot(interp, w1_ref[...], preferred_element_type=jnp.float32)
    h = jnp.maximum(h + b1_ref[...], 0.0)
    out = jnp.dot(h, w2_ref[...], preferred_element_type=jnp.float32)
    o_ref[...] = out + b2_ref[...]


@functools.partial(jax.jit, static_argnames=())
def kernel(vertices, centroids, centroid_features, W1, b1, W2, b2):
    vt = vertices.T
    cx = jnp.broadcast_to(centroids[:, 0:1], (M, TN))
    cy = jnp.broadcast_to(centroids[:, 1:2], (M, TN))
    cz = jnp.broadcast_to(centroids[:, 2:3], (M, TN))
    fpad = jnp.concatenate(
        [centroid_features,
         jnp.ones((M, 1), jnp.float32),
         jnp.zeros((M, 127), jnp.float32)], axis=1)
    w1a = W1[:C, :]
    b1r = b1.reshape(1, H)
    b2r = b2.reshape(1, H)
    grid = (N // TN,)
    out = pl.pallas_call(
        _fused_body,
        grid=grid,
        in_specs=[
            pl.BlockSpec((3, TN), lambda i: (0, i)),
            pl.BlockSpec((M, TN), lambda i: (0, 0)),
            pl.BlockSpec((M, TN), lambda i: (0, 0)),
            pl.BlockSpec((M, TN), lambda i: (0, 0)),
            pl.BlockSpec((M, 2 * C), lambda i: (0, 0)),
            pl.BlockSpec((C, H), lambda i: (0, 0)),
            pl.BlockSpec((1, H), lambda i: (0, 0)),
            pl.BlockSpec((H, H), lambda i: (0, 0)),
            pl.BlockSpec((1, H), lambda i: (0, 0)),
        ],
        out_specs=pl.BlockSpec((TN, H), lambda i: (i, 0)),
        out_shape=jax.ShapeDtypeStruct((N, H), jnp.float32),
    )(vt, cx, cy, cz, fpad, w1a, b1r, W2, b2r)
    return out


# speedup vs baseline: 1.6751x; 1.4747x over previous
import functools

import jax
import jax.numpy as jnp
from jax.experimental import pallas as pl
from jax.experimental.pallas import tpu as pltpu

N = 16384
M = 1024
C = 128
H = 256
TN = 512
WCLAMP = 1e30
_STRIP = 8
_NSTRIP = M // _STRIP


def _insert3(a1, a2, a3, v):
    t = jnp.maximum(a1, v)
    a1 = jnp.minimum(a1, v)
    u = jnp.maximum(a2, t)
    a2 = jnp.minimum(a2, t)
    a3 = jnp.minimum(a3, u)
    return a1, a2, a3


def _merge33(a, b):
    a1, a2, a3 = a
    b1, b2, b3 = b
    c1 = jnp.minimum(a1, b1)
    y1 = jnp.maximum(a1, b1)
    x2 = jnp.minimum(a2, b2)
    y2 = jnp.maximum(a2, b2)
    x3 = jnp.minimum(a3, b3)
    c2 = jnp.minimum(y1, x2)
    z = jnp.maximum(y1, x2)
    c3 = jnp.minimum(jnp.minimum(z, x3), y2)
    return c1, c2, c3


def _third_from_triple(a1, a2, a3):
    inf = jnp.float32(jnp.inf)
    m0 = jnp.min(a1, axis=0, keepdims=True)
    c1 = jnp.where(a1 > m0, a1, inf)
    m1 = jnp.minimum(jnp.min(c1, axis=0, keepdims=True),
                     jnp.min(a2, axis=0, keepdims=True))
    c1b = jnp.where(c1 > m1, c1, inf)
    c2b = jnp.where(a2 > m1, a2, inf)
    m2 = jnp.minimum(
        jnp.minimum(jnp.min(c1b, axis=0, keepdims=True),
                    jnp.min(c2b, axis=0, keepdims=True)),
        jnp.min(a3, axis=0, keepdims=True))
    return m2


def _fused_body(vt_ref, cx_ref, cy_ref, cz_ref, f_ref, w1_ref, b1_ref,
                w2_ref, b2_ref, o_ref, d2_ref):
    vt = vt_ref[...]
    vx = vt[0:1, :]
    vy = vt[1:2, :]
    vz = vt[2:3, :]

    inf8 = jnp.full((_STRIP, TN), jnp.inf, jnp.float32)
    acc = [inf8, inf8, inf8]
    bcc = [inf8, inf8, inf8]
    for r in range(_NSTRIP):
        sl = pl.ds(r * _STRIP, _STRIP)
        dx = cx_ref[sl, :] - vx
        d2r = dx * dx
        dy = cy_ref[sl, :] - vy
        d2r += dy * dy
        dz = cz_ref[sl, :] - vz
        d2r += dz * dz
        d2_ref[sl, :] = d2r
        if r % 2 == 0:
            acc = list(_insert3(*acc, d2r))
        else:
            bcc = list(_insert3(*bcc, d2r))

    a1, a2, a3 = _merge33(acc, bcc)
    m2 = _third_from_triple(a1, a2, a3)

    d2 = d2_ref[...]
    d2sel = jnp.where(d2 <= m2, d2, jnp.float32(jnp.inf))
    wmat = jnp.minimum(1.0 / d2sel, WCLAMP)

    acc_f = jax.lax.dot_general(
        wmat, f_ref[...], (((0,), (0,)), ((), ())),
        preferred_element_type=jnp.float32)
    interp = acc_f[:, :C] / acc_f[:, C:C + 1]

    h = jnp.dot(interp, w1_ref[...], preferred_element_type=jnp.float32)
    h = jnp.maximum(h + b1_ref[...], 0.0)
    out = jnp.dot(h, w2_ref[...], preferred_element_type=jnp.float32)
    o_ref[...] = out + b2_ref[...]


@functools.partial(jax.jit, static_argnames=())
def kernel(vertices, centroids, centroid_features, W1, b1, W2, b2):
    vt = vertices.T
    cx = jnp.broadcast_to(centroids[:, 0:1], (M, TN))
    cy = jnp.broadcast_to(centroids[:, 1:2], (M, TN))
    cz = jnp.broadcast_to(centroids[:, 2:3], (M, TN))
    fpad = jnp.concatenate(
        [centroid_features,
         jnp.ones((M, 1), jnp.float32),
         jnp.zeros((M, 127), jnp.float32)], axis=1)
    w1a = W1[:C, :]
    b1r = b1.reshape(1, H)
    b2r = b2.reshape(1, H)
    grid = (N // TN,)
    out = pl.pallas_call(
        _fused_body,
        grid=grid,
        in_specs=[
            pl.BlockSpec((3, TN), lambda i: (0, i)),
            pl.BlockSpec((M, TN), lambda i: (0, 0)),
            pl.BlockSpec((M, TN), lambda i: (0, 0)),
            pl.BlockSpec((M, TN), lambda i: (0, 0)),
            pl.BlockSpec((M, 2 * C), lambda i: (0, 0)),
            pl.BlockSpec((C, H), lambda i: (0, 0)),
            pl.BlockSpec((1, H), lambda i: (0, 0)),
            pl.BlockSpec((H, H), lambda i: (0, 0)),
            pl.BlockSpec((1, H), lambda i: (0, 0)),
        ],
        out_specs=pl.BlockSpec((TN, H), lambda i: (i, 0)),
        out_shape=jax.ShapeDtypeStruct((N, H), jnp.float32),
        scratch_shapes=[pltpu.VMEM((M, TN), jnp.float32)],
    )(vt, cx, cy, cz, fpad, w1a, b1r, W2, b2r)
    return out


# speedup vs baseline: 1.6835x; 1.0050x over previous
import functools

import jax
import jax.numpy as jnp
from jax.experimental import pallas as pl
from jax.experimental.pallas import tpu as pltpu

N = 16384
M = 1024
C = 128
H = 256
TN = 1024
WCLAMP = 1e30
_STRIP = 8
_NSTRIP = M // _STRIP


def _insert3(a1, a2, a3, v):
    t = jnp.maximum(a1, v)
    a1 = jnp.minimum(a1, v)
    u = jnp.maximum(a2, t)
    a2 = jnp.minimum(a2, t)
    a3 = jnp.minimum(a3, u)
    return a1, a2, a3


def _merge33(a, b):
    a1, a2, a3 = a
    b1, b2, b3 = b
    c1 = jnp.minimum(a1, b1)
    y1 = jnp.maximum(a1, b1)
    x2 = jnp.minimum(a2, b2)
    y2 = jnp.maximum(a2, b2)
    x3 = jnp.minimum(a3, b3)
    c2 = jnp.minimum(y1, x2)
    z = jnp.maximum(y1, x2)
    c3 = jnp.minimum(jnp.minimum(z, x3), y2)
    return c1, c2, c3


def _third_from_triple(a1, a2, a3):
    inf = jnp.float32(jnp.inf)
    m0 = jnp.min(a1, axis=0, keepdims=True)
    c1 = jnp.where(a1 > m0, a1, inf)
    m1 = jnp.minimum(jnp.min(c1, axis=0, keepdims=True),
                     jnp.min(a2, axis=0, keepdims=True))
    c1b = jnp.where(c1 > m1, c1, inf)
    c2b = jnp.where(a2 > m1, a2, inf)
    m2 = jnp.minimum(
        jnp.minimum(jnp.min(c1b, axis=0, keepdims=True),
                    jnp.min(c2b, axis=0, keepdims=True)),
        jnp.min(a3, axis=0, keepdims=True))
    return m2


def _fused_body(vt_ref, cx_ref, cy_ref, cz_ref, f_ref, w1_ref, b1_ref,
                w2_ref, b2_ref, o_ref, d2_ref):
    vt = vt_ref[...]
    vx = vt[0:1, :]
    vy = vt[1:2, :]
    vz = vt[2:3, :]

    inf8 = jnp.full((_STRIP, TN), jnp.inf, jnp.float32)
    accs = [[inf8, inf8, inf8] for _ in range(4)]
    for r in range(_NSTRIP):
        sl = pl.ds(r * _STRIP, _STRIP)
        dx = cx_ref[sl, :] - vx
        d2r = dx * dx
        dy = cy_ref[sl, :] - vy
        d2r += dy * dy
        dz = cz_ref[sl, :] - vz
        d2r += dz * dz
        d2_ref[sl, :] = d2r
        k = r % 4
        accs[k] = list(_insert3(*accs[k], d2r))

    a1, a2, a3 = _merge33(_merge33(accs[0], accs[1]),
                          _merge33(accs[2], accs[3]))
    m2 = _third_from_triple(a1, a2, a3)

    d2 = d2_ref[...]
    d2sel = jnp.where(d2 <= m2, d2, jnp.float32(jnp.inf))
    wmat = jnp.minimum(1.0 / d2sel, WCLAMP)

    acc_f = jax.lax.dot_general(
        wmat, f_ref[...], (((0,), (0,)), ((), ())),
        preferred_element_type=jnp.float32)
    interp = acc_f[:, :C] / acc_f[:, C:C + 1]

    h = jnp.dot(interp, w1_ref[...], preferred_element_type=jnp.float32)
    h = jnp.maximum(h + b1_ref[...], 0.0)
    out = jnp.dot(h, w2_ref[...], preferred_element_type=jnp.float32)
    o_ref[...] = out + b2_ref[...]


@functools.partial(jax.jit, static_argnames=())
def kernel(vertices, centroids, centroid_features, W1, b1, W2, b2):
    vt = vertices.T
    cx = jnp.broadcast_to(centroids[:, 0:1], (M, TN))
    cy = jnp.broadcast_to(centroids[:, 1:2], (M, TN))
    cz = jnp.broadcast_to(centroids[:, 2:3], (M, TN))
    fpad = jnp.concatenate(
        [centroid_features,
         jnp.ones((M, 1), jnp.float32),
         jnp.zeros((M, 127), jnp.float32)], axis=1)
    w1a = W1[:C, :]
    b1r = b1.reshape(1, H)
    b2r = b2.reshape(1, H)
    grid = (N // TN,)
    out = pl.pallas_call(
        _fused_body,
        grid=grid,
        in_specs=[
            pl.BlockSpec((3, TN), lambda i: (0, i)),
            pl.BlockSpec((M, TN), lambda i: (0, 0)),
            pl.BlockSpec((M, TN), lambda i: (0, 0)),
            pl.BlockSpec((M, TN), lambda i: (0, 0)),
            pl.BlockSpec((M, 2 * C), lambda i: (0, 0)),
            pl.BlockSpec((C, H), lambda i: (0, 0)),
            pl.BlockSpec((1, H), lambda i: (0, 0)),
            pl.BlockSpec((H, H), lambda i: (0, 0)),
            pl.BlockSpec((1, H), lambda i: (0, 0)),
        ],
        out_specs=pl.BlockSpec((TN, H), lambda i: (i, 0)),
        out_shape=jax.ShapeDtypeStruct((N, H), jnp.float32),
        scratch_shapes=[pltpu.VMEM((M, TN), jnp.float32)],
    )(vt, cx, cy, cz, fpad, w1a, b1r, W2, b2r)
    return out


# speedup vs baseline: 1.6853x; 1.0011x over previous
import functools

import jax
import jax.numpy as jnp
from jax.experimental import pallas as pl
from jax.experimental.pallas import tpu as pltpu

N = 16384
M = 1024
C = 128
H = 256
TN = 1024
WCLAMP = 1e30
_STRIP = 8
_NSTRIP = M // _STRIP


def _insert3(a1, a2, a3, v):
    t = jnp.maximum(a1, v)
    a1 = jnp.minimum(a1, v)
    u = jnp.maximum(a2, t)
    a2 = jnp.minimum(a2, t)
    a3 = jnp.minimum(a3, u)
    return a1, a2, a3


def _merge33(a, b):
    a1, a2, a3 = a
    b1, b2, b3 = b
    c1 = jnp.minimum(a1, b1)
    y1 = jnp.maximum(a1, b1)
    x2 = jnp.minimum(a2, b2)
    y2 = jnp.maximum(a2, b2)
    x3 = jnp.minimum(a3, b3)
    c2 = jnp.minimum(y1, x2)
    z = jnp.maximum(y1, x2)
    c3 = jnp.minimum(jnp.minimum(z, x3), y2)
    return c1, c2, c3


def _third_from_triple(a1, a2, a3):
    inf = jnp.float32(jnp.inf)
    m0 = jnp.min(a1, axis=0, keepdims=True)
    c1 = jnp.where(a1 > m0, a1, inf)
    m1 = jnp.minimum(jnp.min(c1, axis=0, keepdims=True),
                     jnp.min(a2, axis=0, keepdims=True))
    c1b = jnp.where(c1 > m1, c1, inf)
    c2b = jnp.where(a2 > m1, a2, inf)
    m2 = jnp.minimum(
        jnp.minimum(jnp.min(c1b, axis=0, keepdims=True),
                    jnp.min(c2b, axis=0, keepdims=True)),
        jnp.min(a3, axis=0, keepdims=True))
    return m2


def _fused_body(vt_ref, cx_ref, cy_ref, cz_ref, f_ref, w1_ref, b1_ref,
                w2_ref, b2_ref, o_ref, d2_ref):
    vt = vt_ref[...]
    vx = vt[0:1, :]
    vy = vt[1:2, :]
    vz = vt[2:3, :]

    inf8 = jnp.full((_STRIP, TN), jnp.inf, jnp.float32)
    accs = [[inf8, inf8, inf8] for _ in range(4)]
    for r in range(_NSTRIP):
        sl = pl.ds(r * _STRIP, _STRIP)
        dx = cx_ref[sl, :] - vx
        d2r = dx * dx
        dy = cy_ref[sl, :] - vy
        d2r += dy * dy
        dz = cz_ref[sl, :] - vz
        d2r += dz * dz
        d2_ref[sl, :] = d2r
        k = r % 4
        accs[k] = list(_insert3(*accs[k], d2r))

    a1, a2, a3 = _merge33(_merge33(accs[0], accs[1]),
                          _merge33(accs[2], accs[3]))
    m2 = _third_from_triple(a1, a2, a3)

    d2 = d2_ref[...]
    d2sel = jnp.where(d2 <= m2, d2, jnp.float32(jnp.inf))
    wmat = jnp.minimum(1.0 / d2sel, WCLAMP)

    acc_f = jax.lax.dot_general(
        wmat, f_ref[...], (((0,), (0,)), ((), ())),
        preferred_element_type=jnp.float32)
    interp = acc_f[:, :C] / acc_f[:, C:C + 1]

    h = jnp.dot(interp, w1_ref[...], preferred_element_type=jnp.float32)
    h = jnp.maximum(h + b1_ref[...], 0.0)
    out = jnp.dot(h, w2_ref[...], preferred_element_type=jnp.float32)
    o_ref[...] = out + b2_ref[...]


@functools.partial(jax.jit, static_argnames=())
def kernel(vertices, centroids, centroid_features, W1, b1, W2, b2):
    vt = vertices.T
    cx = jnp.broadcast_to(centroids[:, 0:1], (M, TN))
    cy = jnp.broadcast_to(centroids[:, 1:2], (M, TN))
    cz = jnp.broadcast_to(centroids[:, 2:3], (M, TN))
    fpad = jnp.concatenate(
        [centroid_features,
         jnp.ones((M, 1), jnp.float32),
         jnp.zeros((M, 127), jnp.float32)], axis=1)
    w1a = W1[:C, :]
    b1r = b1.reshape(1, H)
    b2r = b2.reshape(1, H)
    grid = (N // TN,)
    out = pl.pallas_call(
        _fused_body,
        grid=grid,
        in_specs=[
            pl.BlockSpec((3, TN), lambda i: (0, i)),
            pl.BlockSpec(memory_space=pltpu.VMEM),
            pl.BlockSpec(memory_space=pltpu.VMEM),
            pl.BlockSpec(memory_space=pltpu.VMEM),
            pl.BlockSpec(memory_space=pltpu.VMEM),
            pl.BlockSpec(memory_space=pltpu.VMEM),
            pl.BlockSpec(memory_space=pltpu.VMEM),
            pl.BlockSpec(memory_space=pltpu.VMEM),
            pl.BlockSpec(memory_space=pltpu.VMEM),
        ],
        out_specs=pl.BlockSpec((TN, H), lambda i: (i, 0)),
        out_shape=jax.ShapeDtypeStruct((N, H), jnp.float32),
        scratch_shapes=[pltpu.VMEM((M, TN), jnp.float32)],
    )(vt, cx, cy, cz, fpad, w1a, b1r, W2, b2r)
    return out
